# Initial kernel scaffold; baseline (speedup 1.0000x reference)
import jax
import jax.numpy as jnp
from jax.experimental import pallas as pl


def kernel(x, w_gate, W1, b1, W2, b2):
    raise NotImplementedError("write your pallas kernel here")


# speedup vs baseline: 1.2222x; 1.2222x over previous
import functools

import jax
import jax.numpy as jnp
from jax import lax
from jax.experimental import pallas as pl
from jax.experimental.pallas import tpu as pltpu


def _cv_sq(v):
    eps = 1e-10
    return jnp.var(v, ddof=1) / (jnp.mean(v) ** 2 + eps)


def _logits_body(x_ref, wg_ref, o_ref):
    o_ref[...] = jnp.dot(x_ref[...], wg_ref[...],
                         preferred_element_type=jnp.float32)


def _moe_body(x_ref, w1_ref, b1_ref, w2_ref, b2_ref, g_ref, o_ref):
    E = w1_ref.shape[0]
    x = x_ref[...].astype(jnp.bfloat16)
    g = g_ref[...]
    acc = jnp.zeros(o_ref.shape, jnp.float32)
    for e in range(E):
        h = jnp.dot(x, w1_ref[e], preferred_element_type=jnp.float32)
        h = jnp.maximum(h + b1_ref[e][None, :], 0.0).astype(jnp.bfloat16)
        y = jnp.dot(h, w2_ref[e], preferred_element_type=jnp.float32)
        y = y + b2_ref[e][None, :]
        acc = acc + y * g[:, e:e + 1]
    o_ref[...] = acc


def _gating(x, w_gate):
    N, D = x.shape
    E = w_gate.shape[1]
    TN = 256
    logits = pl.pallas_call(
        _logits_body,
        grid=(N // TN,),
        in_specs=[pl.BlockSpec((TN, D), lambda i: (i, 0)),
                  pl.BlockSpec((D, E), lambda i: (0, 0))],
        out_specs=pl.BlockSpec((TN, E), lambda i: (i, 0)),
        out_shape=jax.ShapeDtypeStruct((N, E), jnp.float32),
    )(x, w_gate)
    top_l, top_i = lax.top_k(logits, 2)
    gg = jax.nn.softmax(top_l, axis=-1)
    gates = jnp.zeros((N, E), x.dtype).at[
        jnp.arange(N)[:, None], top_i].set(gg)
    importance = gates.sum(axis=0)
    load = (gates > 0).sum(axis=0).astype(jnp.float32)
    loss = (_cv_sq(importance) + _cv_sq(load)) * 1e-2
    return gates, top_i, gg, loss


def kernel(x, w_gate, W1, b1, W2, b2):
    N, D = x.shape
    E = w_gate.shape[1]
    H = W1.shape[2]
    gates, _, _, loss = _gating(x, w_gate)

    TN = 256
    y = pl.pallas_call(
        _moe_body,
        grid=(N // TN,),
        in_specs=[
            pl.BlockSpec((TN, D), lambda i: (i, 0)),
            pl.BlockSpec((E, D, H), lambda i: (0, 0, 0)),
            pl.BlockSpec((E, H), lambda i: (0, 0)),
            pl.BlockSpec((E, H, D), lambda i: (0, 0, 0)),
            pl.BlockSpec((E, D), lambda i: (0, 0)),
            pl.BlockSpec((TN, E), lambda i: (i, 0)),
        ],
        out_specs=pl.BlockSpec((TN, D), lambda i: (i, 0)),
        out_shape=jax.ShapeDtypeStruct((N, D), jnp.float32),
    )(x, W1.astype(jnp.bfloat16), b1, W2.astype(jnp.bfloat16), b2, gates)
    return y, loss
